# baseline (speedup 1.0000x reference)
import jax
import jax.numpy as jnp
from jax.experimental import pallas as pl
from jax.experimental.pallas import tpu as pltpu

B = 8192
D = 256
K = 1024
E = 8
BETA = 1.0

BM = 2048
NB = B // BM

_HI = jax.lax.Precision.HIGHEST
_DEF = jax.lax.Precision.DEFAULT


def _rvq_body(x_ref, cb0_ref, cb1_ref, cb2_ref, cbn0_ref, cbn1_ref,
              cbn2_ref, gw_ref, gb_ref,
              xq_ref, idx_ref, loss_ref,
              hi_ref, mid_ref, lo_ref, csum_ref):
    @pl.when(pl.program_id(0) == 0)
    def _prep():
        for s, cb_ref in enumerate((cb0_ref, cb1_ref, cb2_ref)):
            cb = cb_ref[...]
            csum_ref[s, :] = jnp.sum(cb * cb, axis=1)
            hi = cb.astype(jnp.bfloat16)
            rem = cb - hi.astype(jnp.float32)
            mid = rem.astype(jnp.bfloat16)
            lo = (rem - mid.astype(jnp.float32)).astype(jnp.bfloat16)
            hi_ref[s, :, :] = hi
            mid_ref[s, :, :] = mid
            lo_ref[s, :, :] = lo

    x = x_ref[...]
    r = x
    xq = jnp.zeros_like(x)
    idx_cols = []
    losses = []
    for s, cbn_ref in enumerate((cbn0_ref, cbn1_ref, cbn2_ref)):
        csum = csum_ref[s, :]
        rsum = jnp.sum(r * r, axis=1)
        m2 = jax.lax.dot_general(r, cbn_ref[...], (((1,), (1,)), ((), ())),
                                 precision=_DEF)
        d = (rsum[:, None] + m2) + csum[None, :]
        dmin = jnp.min(d, axis=1, keepdims=True)
        iota = jax.lax.broadcasted_iota(jnp.int32, (BM, K), 1)
        idx = jnp.min(jnp.where(d == dmin, iota, K), axis=1).astype(jnp.int32)
        onehot = (idx[:, None] == iota).astype(jnp.bfloat16)
        hi = hi_ref[s, :, :]
        mid = mid_ref[s, :, :]
        lo = lo_ref[s, :, :]
        dims = (((1,), (0,)), ((), ()))
        q = ((jax.lax.dot_general(onehot, hi, dims,
                                  preferred_element_type=jnp.float32)
              + jax.lax.dot_general(onehot, mid, dims,
                                    preferred_element_type=jnp.float32))
             + jax.lax.dot_general(onehot, lo, dims,
                                   preferred_element_type=jnp.float32))
        diff = q - r
        losses.append(jnp.sum(diff * diff))
        x_res = r + diff
        r = r - x_res
        xq = xq + x_res
        idx_cols.append(idx)
    logits = jax.lax.dot_general(x, gw_ref[...], (((1,), (0,)), ((), ())),
                                 precision=_DEF) + gb_ref[...]
    lmax = jnp.max(logits, axis=-1, keepdims=True)
    iota_e = jax.lax.broadcasted_iota(jnp.int32, (BM, E), 1)
    expert = jnp.min(jnp.where(logits == lmax, iota_e, E),
                     axis=1).astype(jnp.int32)
    idx_cols.append(expert)

    xq_ref[...] = xq
    idx_ref[...] = jnp.stack(idx_cols, axis=-1)
    loss_ref[...] = jnp.stack(losses).reshape(1, 1, 3)


def kernel(x, codebook_0, codebook_1, codebook_2, gate_W, gate_b,
           labels_0, labels_1, labels_2):
    del labels_0, labels_1, labels_2
    gate_b2 = gate_b.reshape(1, E)
    xq, idx, loss_parts = pl.pallas_call(
        _rvq_body,
        grid=(NB,),
        in_specs=[
            pl.BlockSpec((BM, D), lambda i: (i, 0)),
            pl.BlockSpec((K, D), lambda i: (0, 0)),
            pl.BlockSpec((K, D), lambda i: (0, 0)),
            pl.BlockSpec((K, D), lambda i: (0, 0)),
            pl.BlockSpec((K, D), lambda i: (0, 0)),
            pl.BlockSpec((K, D), lambda i: (0, 0)),
            pl.BlockSpec((K, D), lambda i: (0, 0)),
            pl.BlockSpec((D, E), lambda i: (0, 0)),
            pl.BlockSpec((1, E), lambda i: (0, 0)),
        ],
        out_specs=[
            pl.BlockSpec((BM, D), lambda i: (i, 0)),
            pl.BlockSpec((BM, 4), lambda i: (i, 0)),
            pl.BlockSpec((1, 1, 3), lambda i: (i, 0, 0)),
        ],
        out_shape=[
            jax.ShapeDtypeStruct((B, D), jnp.float32),
            jax.ShapeDtypeStruct((B, 4), jnp.int32),
            jax.ShapeDtypeStruct((NB, 1, 3), jnp.float32),
        ],
        scratch_shapes=[
            pltpu.VMEM((3, K, D), jnp.bfloat16),
            pltpu.VMEM((3, K, D), jnp.bfloat16),
            pltpu.VMEM((3, K, D), jnp.bfloat16),
            pltpu.VMEM((3, K), jnp.float32),
        ],
    )(x, codebook_0, codebook_1, codebook_2,
      -2.0 * codebook_0, -2.0 * codebook_1, -2.0 * codebook_2,
      gate_W, gate_b2)
    mean_losses = jnp.sum(loss_parts) * ((1.0 + BETA) / (3.0 * B * D))
    return (xq, mean_losses, idx)


# speedup vs baseline: 1.0528x; 1.0528x over previous
import jax
import jax.numpy as jnp
from jax.experimental import pallas as pl

B = 8192
D = 256
K = 1024
E = 8
BETA = 1.0

BM = 2048
NB = B // BM

_HI = jax.lax.Precision.HIGHEST
_DEF = jax.lax.Precision.DEFAULT


def _rvq_body(x_ref, cb0_ref, cb1_ref, cb2_ref, gw_ref, gb_ref,
              xq_ref, idx_ref, loss_ref):
    x = x_ref[...]
    r = x
    xq = jnp.zeros_like(x)
    idx_cols = []
    losses = []
    for cb_ref in (cb0_ref, cb1_ref, cb2_ref):
        cb = cb_ref[...]
        csum = jnp.sum(cb * cb, axis=1)
        rsum = jnp.sum(r * r, axis=1)
        m = jax.lax.dot_general(r, cb, (((1,), (1,)), ((), ())),
                                precision=_DEF)
        d = (rsum[:, None] - 2.0 * m) + csum[None, :]
        dmin = jnp.min(d, axis=1, keepdims=True)
        iota = jax.lax.broadcasted_iota(jnp.int32, (BM, K), 1)
        idx = jnp.min(jnp.where(d == dmin, iota, K), axis=1).astype(jnp.int32)
        onehot = (idx[:, None] == iota).astype(jnp.bfloat16)
        hi = cb.astype(jnp.bfloat16)
        rem = cb - hi.astype(jnp.float32)
        mid = rem.astype(jnp.bfloat16)
        lo = (rem - mid.astype(jnp.float32)).astype(jnp.bfloat16)
        dims = (((1,), (0,)), ((), ()))
        q = ((jax.lax.dot_general(onehot, hi, dims,
                                  preferred_element_type=jnp.float32)
              + jax.lax.dot_general(onehot, mid, dims,
                                    preferred_element_type=jnp.float32))
             + jax.lax.dot_general(onehot, lo, dims,
                                   preferred_element_type=jnp.float32))
        diff = q - r
        losses.append(jnp.sum(diff * diff))
        x_res = r + diff
        r = r - x_res
        xq = xq + x_res
        idx_cols.append(idx)
    logits = jax.lax.dot_general(x, gw_ref[...], (((1,), (0,)), ((), ())),
                                 precision=_DEF) + gb_ref[...]
    lmax = jnp.max(logits, axis=-1, keepdims=True)
    iota_e = jax.lax.broadcasted_iota(jnp.int32, (BM, E), 1)
    expert = jnp.min(jnp.where(logits == lmax, iota_e, E),
                     axis=1).astype(jnp.int32)
    idx_cols.append(expert)

    xq_ref[...] = xq
    idx_ref[...] = jnp.stack(idx_cols, axis=-1)
    loss_ref[...] = jnp.stack(losses).reshape(1, 1, 3)


def kernel(x, codebook_0, codebook_1, codebook_2, gate_W, gate_b,
           labels_0, labels_1, labels_2):
    del labels_0, labels_1, labels_2
    gate_b2 = gate_b.reshape(1, E)
    xq, idx, loss_parts = pl.pallas_call(
        _rvq_body,
        grid=(NB,),
        in_specs=[
            pl.BlockSpec((BM, D), lambda i: (i, 0)),
            pl.BlockSpec((K, D), lambda i: (0, 0)),
            pl.BlockSpec((K, D), lambda i: (0, 0)),
            pl.BlockSpec((K, D), lambda i: (0, 0)),
            pl.BlockSpec((D, E), lambda i: (0, 0)),
            pl.BlockSpec((1, E), lambda i: (0, 0)),
        ],
        out_specs=[
            pl.BlockSpec((BM, D), lambda i: (i, 0)),
            pl.BlockSpec((BM, 4), lambda i: (i, 0)),
            pl.BlockSpec((1, 1, 3), lambda i: (i, 0, 0)),
        ],
        out_shape=[
            jax.ShapeDtypeStruct((B, D), jnp.float32),
            jax.ShapeDtypeStruct((B, 4), jnp.int32),
            jax.ShapeDtypeStruct((NB, 1, 3), jnp.float32),
        ],
    )(x, codebook_0, codebook_1, codebook_2, gate_W, gate_b2)
    mean_losses = jnp.sum(loss_parts) * ((1.0 + BETA) / (3.0 * B * D))
    return (xq, mean_losses, idx)
